# baseline (speedup 1.0000x reference)
import functools

import jax
import jax.numpy as jnp
from jax import lax
from jax.experimental import pallas as pl
from jax.experimental.pallas import tpu as pltpu
from jax.experimental.pallas import tpu_sc as plsc

_NC = 2
_NS = 16
_NW = _NC * _NS
_LANES = 16
_CHUNK = 100


def _position_encoding(seq_len, hidden, start, dtype):
    power = jnp.arange(0, hidden, 2, dtype=dtype) / hidden
    divisor = 10000.0 ** power
    seqpos = jnp.arange(start, seq_len + start, dtype=dtype)
    index = seqpos[:, None] / divisor[None, :]
    pe = jnp.stack((jnp.sin(index), jnp.cos(index)), axis=-1)
    return pe.reshape(seq_len, hidden)


def _body(idx_hbm, table_hbm, pe_hbm, out_hbm, idx_v, rows_v, pe_v, sem_in):
    nb = idx_hbm.shape[0] // _NW
    seq = pe_hbm.shape[0]
    wid = lax.axis_index("s") * _NC + lax.axis_index("c")
    base = wid * nb

    pltpu.sync_copy(pe_hbm, pe_v)

    def batch_body(i, carry):
        b = base + i
        pltpu.sync_copy(idx_hbm.at[b], idx_v)
        cp0 = pltpu.async_copy(
            table_hbm.at[idx_v.at[0]], rows_v.at[pl.ds(0, _CHUNK)], sem_in
        )
        cp1 = pltpu.async_copy(
            table_hbm.at[idx_v.at[1]], rows_v.at[pl.ds(_CHUNK, _CHUNK)], sem_in
        )
        cp0.wait()
        cp1.wait()

        def row_body(r, c2):
            for c in range(4):
                sl = pl.ds(c * _LANES, _LANES)
                rows_v[r, sl] = rows_v[r, sl] + pe_v[r, sl]
            return c2

        lax.fori_loop(0, seq, row_body, 0)
        pltpu.sync_copy(rows_v, out_hbm.at[b])
        return carry

    lax.fori_loop(0, nb, batch_body, 0)


def kernel(inputs, embedding_table):
    batch, seq = inputs.shape
    _, embed = embedding_table.shape
    pe = _position_encoding(seq, embed, 1, embedding_table.dtype)
    idx = inputs.reshape(batch, seq // _CHUNK, _CHUNK)

    mesh = plsc.VectorSubcoreMesh(
        core_axis_name="c", subcore_axis_name="s", num_cores=_NC, num_subcores=_NS
    )
    run = pl.kernel(
        _body,
        out_type=jax.ShapeDtypeStruct((batch, seq, embed), embedding_table.dtype),
        mesh=mesh,
        scratch_types=[
            pltpu.VMEM((seq // _CHUNK, _CHUNK), jnp.int32),
            pltpu.VMEM((seq, embed), jnp.float32),
            pltpu.VMEM((seq, embed), jnp.float32),
            pltpu.SemaphoreType.DMA,
        ],
        compiler_params=pltpu.CompilerParams(use_tc_tiling_on_sc=False),
    )
    return run(idx, embedding_table, pe)
